# Initial kernel scaffold; baseline (speedup 1.0000x reference)
import jax
import jax.numpy as jnp
from jax.experimental import pallas as pl


def kernel(x, ln_in_g, ln_in_b, ln_r_g, ln_r_b, W_r, b_r, e_ln1_g, e_ln1_b, e_W1, e_b1, e_ln2_g, e_ln2_b, e_W2, e_b2, e_ln3_g, e_ln3_b, e_W3, e_b3):
    raise NotImplementedError("write your pallas kernel here")


# speedup vs baseline: 3.1697x; 3.1697x over previous
import functools
import math

import jax
import jax.numpy as jnp
from jax.experimental import pallas as pl

EPS = 1e-5
_NEG = -3.4e38


def _normalize(x):
    m = jnp.mean(x, axis=-1, keepdims=True)
    v = jnp.mean((x - m) ** 2, axis=-1, keepdims=True)
    return (x - m) * jax.lax.rsqrt(v + EPS)


def _gelu(x):
    return x * 0.5 * (1.0 + jax.lax.erf(x * (1.0 / math.sqrt(2.0))))


def _moe_block(ln_in_g_r, ln_in_b_r, Wr_r, cr_r, W1_r, c1_r, W2_r, c2_r,
               W3_r, c3_r, x_r, final_r, logits_r, frac_r, prob_r):
    i = pl.program_id(0)
    nsteps = pl.num_programs(0)
    E = W1_r.shape[0]

    x = x_r[...]
    xn = _normalize(x) * ln_in_g_r[...] + ln_in_b_r[...]
    znx = _normalize(xn)

    logits = jnp.dot(znx, Wr_r[...], preferred_element_type=jnp.float32)
    logits = logits + cr_r[...]
    logits_r[...] = logits

    iota = jax.lax.broadcasted_iota(jnp.int32, logits.shape, 1)
    v1 = jnp.max(logits, axis=-1, keepdims=True)
    i1 = jnp.min(jnp.where(logits == v1, iota, E), axis=-1, keepdims=True)
    masked = jnp.where(iota == i1, _NEG, logits)
    v2 = jnp.max(masked, axis=-1, keepdims=True)
    i2 = jnp.min(jnp.where(masked == v2, iota, E), axis=-1, keepdims=True)
    t = jnp.exp(v2 - v1)
    w1 = 1.0 / (1.0 + t)
    w2 = t / (1.0 + t)
    sparse_w = jnp.where(iota == i1, w1, 0.0) + jnp.where(iota == i2, w2, 0.0)

    routed = (iota == i1).astype(jnp.float32) + (iota == i2).astype(jnp.float32)
    sm = jnp.exp(logits - v1)
    sm = sm / jnp.sum(sm, axis=-1, keepdims=True)

    @pl.when(i == 0)
    def _():
        frac_r[...] = jnp.zeros_like(frac_r)
        prob_r[...] = jnp.zeros_like(prob_r)

    frac_r[...] += jnp.sum(routed, axis=0, keepdims=True)
    prob_r[...] += jnp.sum(sm, axis=0, keepdims=True)

    @pl.when(i == nsteps - 1)
    def _():
        n_total = nsteps * x.shape[0]
        frac_r[...] *= 1.0 / n_total
        prob_r[...] *= 1.0 / n_total

    zb = znx.astype(jnp.bfloat16)
    final = jnp.zeros((x.shape[0],), jnp.float32)
    for e in range(E):
        h1 = jnp.dot(zb, W1_r[e], preferred_element_type=jnp.float32)
        h1 = _gelu(h1 + c1_r[e])
        n1 = _normalize(h1).astype(jnp.bfloat16)
        h2 = jnp.dot(n1, W2_r[e], preferred_element_type=jnp.float32)
        h2 = _gelu(h2 + c2_r[e])
        n2 = _normalize(h2)
        out_e = jnp.sum(n2 * W3_r[e], axis=-1) + c3_r[e, 0]
        final = final + out_e * sparse_w[:, e]
    final_r[...] = final[:, None]


def kernel(x, ln_in_g, ln_in_b, ln_r_g, ln_r_b, W_r, b_r, e_ln1_g, e_ln1_b,
           e_W1, e_b1, e_ln2_g, e_ln2_b, e_W2, e_b2, e_ln3_g, e_ln3_b,
           e_W3, e_b3):
    N, D = x.shape
    E = e_W1.shape[0]
    H = e_W1.shape[2]
    H2 = e_W2.shape[2]
    B = 512 if N % 512 == 0 else N
    grid = (N // B,)

    Wr_f = ln_r_g[:, None] * W_r
    cr = ln_r_b @ W_r + b_r
    W1_f = (e_ln1_g[:, :, None] * e_W1).astype(jnp.bfloat16)
    c1 = jnp.einsum("ed,edh->eh", e_ln1_b, e_W1) + e_b1
    W2_f = (e_ln2_g[:, :, None] * e_W2).astype(jnp.bfloat16)
    c2 = jnp.einsum("eh,ehk->ek", e_ln2_b, e_W2) + e_b2
    W3_f = e_ln3_g * e_W3[:, :, 0]
    c3 = (jnp.sum(e_ln3_b * e_W3[:, :, 0], axis=-1, keepdims=True)
          + e_b3)

    full = lambda *s: pl.BlockSpec(s, lambda i: (0,) * len(s))
    out_shapes = (
        jax.ShapeDtypeStruct((N, 1), jnp.float32),
        jax.ShapeDtypeStruct((N, E), jnp.float32),
        jax.ShapeDtypeStruct((1, E), jnp.float32),
        jax.ShapeDtypeStruct((1, E), jnp.float32),
    )
    final, logits, frac, prob = pl.pallas_call(
        _moe_block,
        grid=grid,
        in_specs=[
            full(1, D), full(1, D), full(D, E), full(1, E),
            full(E, D, H), full(E, 1, H), full(E, H, H2), full(E, 1, H2),
            full(E, 1, H2), full(E, 1),
            pl.BlockSpec((B, D), lambda i: (i, 0)),
        ],
        out_specs=[
            pl.BlockSpec((B, 1), lambda i: (i, 0)),
            pl.BlockSpec((B, E), lambda i: (i, 0)),
            pl.BlockSpec((1, E), lambda i: (0, 0)),
            pl.BlockSpec((1, E), lambda i: (0, 0)),
        ],
        out_shape=out_shapes,
    )(ln_in_g[None], ln_in_b[None], Wr_f, cr[None], W1_f, c1[:, None],
      W2_f, c2[:, None], W3_f[:, None], c3, x)
    return (final, frac[0], prob[0], logits)
